# baseline (speedup 1.0000x reference)
import functools
import math

import jax
import jax.numpy as jnp
from jax import lax
from jax.experimental import pallas as pl
from jax.experimental.pallas import tpu as pltpu


def _emd_body(x1_ref, x2_ref, out_ref, *, n_iters):
    x1 = x1_ref[0]
    x2 = x2_ref[0]
    ab = lax.dot_general(
        x2, x1, (((0,), (0,)), ((), ())), preferred_element_type=jnp.float32
    )
    aa = jnp.sum(x1 * x1, axis=0)[None, :]
    bb = jnp.sum(x2 * x2, axis=0)[:, None]
    sqd = jnp.maximum(aa + bb - 2.0 * ab, 0.0)

    n = sqd.shape[1]
    m = sqd.shape[0]
    factorl = float(max(n, m) // n)
    factorr = float(max(n, m) // m)

    def body(i, carry):
        match, satl, satr, level2 = carry
        lvl2 = jnp.where(i == n_iters - 1, 0.0, level2)
        e = jnp.exp2(lvl2 * sqd)
        w1 = e * satr
        s = jnp.sum(w1, axis=0, keepdims=True) + 1e-9
        w2 = w1 * (satl / s)
        ss = jnp.sum(w2, axis=1, keepdims=True) + 1e-9
        r = jnp.minimum(satr / ss, 1.0)
        w3 = w2 * r
        srow = jnp.sum(w3, axis=0, keepdims=True)
        scol = r * (ss - 1e-9)
        satl = jnp.maximum(satl - srow, 0.0)
        satr = jnp.maximum(satr - scol, 0.0)
        match = match + w3
        return match, satl, satr, level2 * 0.25

    match0 = jnp.zeros((m, n), dtype=jnp.float32)
    satl0 = jnp.full((1, n), factorl, dtype=jnp.float32)
    satr0 = jnp.full((m, 1), factorr, dtype=jnp.float32)
    level2_0 = jnp.float32(-(4.0**8) * math.log2(math.e))
    match, _, _, _ = lax.fori_loop(
        0, n_iters, body, (match0, satl0, satr0, level2_0)
    )
    d = jnp.sqrt(jnp.maximum(sqd, 1e-12))
    out_ref[0] = jnp.sum(match * d, keepdims=True)


def kernel(input1, input2):
    B, n, _ = input1.shape
    m = input2.shape[1]
    x1t = input1.transpose(0, 2, 1)
    x2t = input2.transpose(0, 2, 1)
    out = pl.pallas_call(
        functools.partial(_emd_body, n_iters=11),
        grid=(B,),
        in_specs=[
            pl.BlockSpec((1, 3, n), lambda b: (b, 0, 0)),
            pl.BlockSpec((1, 3, m), lambda b: (b, 0, 0)),
        ],
        out_specs=pl.BlockSpec((1, 1, 1), lambda b: (b, 0, 0)),
        out_shape=jax.ShapeDtypeStruct((B, 1, 1), jnp.float32),
        compiler_params=pltpu.CompilerParams(
            dimension_semantics=("arbitrary",),
        ),
    )(x1t, x2t)
    return out[:, 0, 0]


# speedup vs baseline: 1.1861x; 1.1861x over previous
import functools
import math

import jax
import jax.numpy as jnp
from jax import lax
from jax.experimental import pallas as pl
from jax.experimental.pallas import tpu as pltpu


def _emd_body(x1_ref, x2_ref, out_ref, *, n_iters):
    x1 = x1_ref[0]
    x2 = x2_ref[0]
    ab = lax.dot_general(
        x1, x2, (((0,), (0,)), ((), ())), preferred_element_type=jnp.float32
    )
    aa = jnp.sum(x1 * x1, axis=0)[:, None]
    bb = jnp.sum(x2 * x2, axis=0)[None, :]
    sqd = jnp.maximum(aa + bb - 2.0 * ab, 0.0)

    n = sqd.shape[0]
    m = sqd.shape[1]
    factorl = float(max(n, m) // n)
    factorr = float(max(n, m) // m)

    def body(i, carry):
        match, satl, satr, level2 = carry
        lvl2 = jnp.where(i == n_iters - 1, 0.0, level2)
        e = jnp.exp2(lvl2 * sqd)
        w1 = e * satr
        s = jnp.sum(w1, axis=1, keepdims=True) + 1e-9
        w2 = w1 * (satl / s)
        ss = jnp.sum(w2, axis=0, keepdims=True) + 1e-9
        r = jnp.minimum(satr / ss, 1.0)
        w3 = w2 * r
        srow = jnp.sum(w3, axis=1, keepdims=True)
        scol = r * (ss - 1e-9)
        satl = jnp.maximum(satl - srow, 0.0)
        satr = jnp.maximum(satr - scol, 0.0)
        match = match + w3
        return match, satl, satr, level2 * 0.25

    match0 = jnp.zeros((n, m), dtype=jnp.float32)
    satl0 = jnp.full((n, 1), factorl, dtype=jnp.float32)
    satr0 = jnp.full((1, m), factorr, dtype=jnp.float32)
    level2_0 = jnp.float32(-(4.0**8) * math.log2(math.e))
    match, _, _, _ = lax.fori_loop(
        0, n_iters, body, (match0, satl0, satr0, level2_0)
    )
    d = jnp.sqrt(jnp.maximum(sqd, 1e-12))
    out_ref[0] = jnp.sum(match * d, keepdims=True)


def kernel(input1, input2):
    B, n, _ = input1.shape
    m = input2.shape[1]
    x1t = input1.transpose(0, 2, 1)
    x2t = input2.transpose(0, 2, 1)
    out = pl.pallas_call(
        functools.partial(_emd_body, n_iters=11),
        grid=(B,),
        in_specs=[
            pl.BlockSpec((1, 3, n), lambda b: (b, 0, 0)),
            pl.BlockSpec((1, 3, m), lambda b: (b, 0, 0)),
        ],
        out_specs=pl.BlockSpec((1, 1, 1), lambda b: (b, 0, 0)),
        out_shape=jax.ShapeDtypeStruct((B, 1, 1), jnp.float32),
        compiler_params=pltpu.CompilerParams(
            dimension_semantics=("arbitrary",),
        ),
    )(x1t, x2t)
    return out[:, 0, 0]


# speedup vs baseline: 1.2663x; 1.0676x over previous
import functools
import math

import jax
import jax.numpy as jnp
from jax import lax
from jax.experimental import pallas as pl
from jax.experimental.pallas import tpu as pltpu


def _emd_body(x1_ref, x2_ref, out_ref, *, n_iters):
    x1 = x1_ref[0]
    x2 = x2_ref[0]
    ab = lax.dot_general(
        x1, x2, (((0,), (0,)), ((), ())), preferred_element_type=jnp.float32
    )
    aa = jnp.sum(x1 * x1, axis=0)[:, None]
    bb = jnp.sum(x2 * x2, axis=0)[None, :]
    sqd = jnp.maximum(aa + bb - 2.0 * ab, 0.0)

    n = sqd.shape[0]
    m = sqd.shape[1]
    factorl = float(max(n, m) // n)
    factorr = float(max(n, m) // m)

    def body(i, carry):
        match, satl, satr, level2 = carry
        e = jnp.exp2(level2 * sqd)
        w1 = e * satr
        s = jnp.sum(w1, axis=1, keepdims=True) + 1e-9
        w2 = w1 * (satl / s)
        ss = jnp.sum(w2, axis=0, keepdims=True) + 1e-9
        r = jnp.minimum(satr / ss, 1.0)
        w3 = w2 * r
        srow = jnp.sum(w3, axis=1, keepdims=True)
        scol = r * (ss - 1e-9)
        satl = jnp.maximum(satl - srow, 0.0)
        satr = jnp.maximum(satr - scol, 0.0)
        match = match + w3
        return match, satl, satr, level2 * 0.25

    match0 = jnp.zeros((n, m), dtype=jnp.float32)
    satl0 = jnp.full((n, 1), factorl, dtype=jnp.float32)
    satr0 = jnp.full((1, m), factorr, dtype=jnp.float32)
    level2_0 = jnp.float32(-(4.0**8) * math.log2(math.e))
    match, satl, satr, _ = lax.fori_loop(
        0, n_iters - 1, body, (match0, satl0, satr0, level2_0)
    )
    d = jnp.sqrt(jnp.maximum(sqd, 1e-12))
    s = jnp.sum(satr) + 1e-9
    rowfac = satl / s
    ss = satr * (jnp.sum(satl) / s) + 1e-9
    r = jnp.minimum(satr / ss, 1.0)
    cw = satr * r
    t = jnp.sum(d * cw, axis=1, keepdims=True)
    cost_final = jnp.sum(rowfac * t, keepdims=True)
    out_ref[0] = jnp.sum(match * d, keepdims=True) + cost_final


def kernel(input1, input2):
    B, n, _ = input1.shape
    m = input2.shape[1]
    x1t = input1.transpose(0, 2, 1)
    x2t = input2.transpose(0, 2, 1)
    out = pl.pallas_call(
        functools.partial(_emd_body, n_iters=11),
        grid=(B,),
        in_specs=[
            pl.BlockSpec((1, 3, n), lambda b: (b, 0, 0)),
            pl.BlockSpec((1, 3, m), lambda b: (b, 0, 0)),
        ],
        out_specs=pl.BlockSpec((1, 1, 1), lambda b: (b, 0, 0)),
        out_shape=jax.ShapeDtypeStruct((B, 1, 1), jnp.float32),
        compiler_params=pltpu.CompilerParams(
            dimension_semantics=("arbitrary",),
        ),
    )(x1t, x2t)
    return out[:, 0, 0]


# speedup vs baseline: 1.5754x; 1.2441x over previous
import functools
import math

import jax
import jax.numpy as jnp
from jax import lax
from jax.experimental import pallas as pl
from jax.experimental.pallas import tpu as pltpu


def _emd_body(x1_ref, x2_ref, out_ref, *, n_iters):
    x1 = x1_ref[0]
    x2 = x2_ref[0]
    ab = lax.dot_general(
        x1, x2, (((0,), (0,)), ((), ())), preferred_element_type=jnp.float32
    )
    aa = jnp.sum(x1 * x1, axis=0)[:, None]
    bb = jnp.sum(x2 * x2, axis=0)[None, :]
    sqd = jnp.maximum(aa + bb - 2.0 * ab, 0.0)

    n = sqd.shape[0]
    m = sqd.shape[1]
    factorl = float(max(n, m) // n)
    factorr = float(max(n, m) // m)

    def body(i, carry):
        match, satl, satr, level2 = carry
        e = jnp.exp2(level2 * sqd)
        w1 = e * satr
        s = jnp.sum(w1, axis=1, keepdims=True) + 1e-9
        w2 = w1 * (satl / s)
        ss = jnp.sum(w2, axis=0, keepdims=True) + 1e-9
        r = jnp.minimum(satr / ss, 1.0)
        w3 = w2 * r
        srow = jnp.sum(w3, axis=1, keepdims=True)
        scol = r * (ss - 1e-9)
        satl = jnp.maximum(satl - srow, 0.0)
        satr = jnp.maximum(satr - scol, 0.0)
        match = match + w3
        return match, satl, satr, level2 * 0.25

    match0 = jnp.zeros((n, m), dtype=jnp.float32)
    satl0 = jnp.full((n, 1), factorl, dtype=jnp.float32)
    satr0 = jnp.full((1, m), factorr, dtype=jnp.float32)
    level2_0 = jnp.float32(-(4.0**8) * math.log2(math.e))
    match, satl, satr, _ = lax.fori_loop(
        0, n_iters - 1, body, (match0, satl0, satr0, level2_0), unroll=5
    )
    d = jnp.sqrt(jnp.maximum(sqd, 1e-12))
    s = jnp.sum(satr) + 1e-9
    rowfac = satl / s
    ss = satr * (jnp.sum(satl) / s) + 1e-9
    r = jnp.minimum(satr / ss, 1.0)
    cw = satr * r
    t = jnp.sum(d * cw, axis=1, keepdims=True)
    cost_final = jnp.sum(rowfac * t, keepdims=True)
    out_ref[0] = jnp.sum(match * d, keepdims=True) + cost_final


def kernel(input1, input2):
    B, n, _ = input1.shape
    m = input2.shape[1]
    x1t = input1.transpose(0, 2, 1)
    x2t = input2.transpose(0, 2, 1)
    out = pl.pallas_call(
        functools.partial(_emd_body, n_iters=11),
        grid=(B,),
        in_specs=[
            pl.BlockSpec((1, 3, n), lambda b: (b, 0, 0)),
            pl.BlockSpec((1, 3, m), lambda b: (b, 0, 0)),
        ],
        out_specs=pl.BlockSpec((1, 1, 1), lambda b: (b, 0, 0)),
        out_shape=jax.ShapeDtypeStruct((B, 1, 1), jnp.float32),
        compiler_params=pltpu.CompilerParams(
            dimension_semantics=("arbitrary",),
        ),
    )(x1t, x2t)
    return out[:, 0, 0]
